# baseline (speedup 1.0000x reference)
import functools

import jax
import jax.numpy as jnp
from jax import lax
from jax.experimental import pallas as pl
from jax.experimental.pallas import tpu as pltpu
from jax.experimental.pallas import tpu_sc as plsc

D_MODEL = 1024
SCALE = 32.0

_NC = 2
_NS = 16
_NW = _NC * _NS

_B = 4 * 4096
_BPW = _B // _NW
_C = 16
_NCHUNK = _BPW // _C
_NBUF = 6
_LANES = 16
_SLICES_PER_ROW = D_MODEL // _LANES


def _emb_body(x_hbm, table_hbm, out_hbm, idx_v, rows_v, *sems):
    gsems = list(sems[:_NBUF])
    ssems = list(sems[_NBUF:])
    sid = lax.axis_index("s")
    wid = sid * _NC + lax.axis_index("c")
    base = wid * _BPW

    pltpu.sync_copy(x_hbm.at[pl.ds(base, _BPW)], idx_v)

    def start_gather(c, b):
        return pltpu.async_copy(
            table_hbm.at[idx_v.at[pl.ds(c * _C, _C)]], rows_v.at[b], gsems[b])

    def start_scatter(c, b):
        return pltpu.async_copy(
            rows_v.at[b], out_hbm.at[pl.ds(base + c * _C, _C)], ssems[b])

    def scale_chunk(b):
        def row_body(r, carry):
            for j in range(_SLICES_PER_ROW):
                sl = pl.ds(j * _LANES, _LANES)
                rows_v[b, r, sl] = rows_v[b, r, sl] * SCALE
            return carry
        lax.fori_loop(0, _C, row_body, 0)

    gathers = [None] * _NCHUNK
    scatters = [None] * _NCHUNK
    for c in range(_NBUF - 1):
        gathers[c] = start_gather(c, c % _NBUF)

    for c in range(_NCHUNK):
        b = c % _NBUF
        gathers[c].wait()
        scale_chunk(b)
        scatters[c] = start_scatter(c, b)
        nc = c + _NBUF - 1
        if nc < _NCHUNK:
            if nc - _NBUF >= 0:
                scatters[nc - _NBUF].wait()
            gathers[nc] = start_gather(nc, nc % _NBUF)

    for c in range(_NCHUNK - _NBUF, _NCHUNK):
        scatters[c].wait()


@jax.jit
def kernel(x, table):
    xf = x.reshape(-1).astype(jnp.int32)

    mesh = plsc.VectorSubcoreMesh(core_axis_name="c", subcore_axis_name="s")
    run = functools.partial(
        pl.kernel,
        mesh=mesh,
        out_type=jax.ShapeDtypeStruct((_B, D_MODEL), jnp.float32),
        scratch_types=[
            pltpu.VMEM((_BPW,), jnp.int32),
            pltpu.VMEM((_NBUF, _C, D_MODEL), jnp.float32),
        ] + [pltpu.SemaphoreType.DMA] * (2 * _NBUF),
    )(_emb_body)
    out = run(xf, table)
    return out.reshape(x.shape + (D_MODEL,))


# speedup vs baseline: 1.0571x; 1.0571x over previous
import functools

import jax
import jax.numpy as jnp
from jax import lax
from jax.experimental import pallas as pl
from jax.experimental.pallas import tpu as pltpu
from jax.experimental.pallas import tpu_sc as plsc

D_MODEL = 1024
SCALE = 32.0

_NC = 2
_NS = 16
_NW = _NC * _NS

_B = 4 * 4096
_BPW = _B // _NW
_C = 32
_NCHUNK = _BPW // _C
_NBUF = 3
_LANES = 16
_SLICES_PER_ROW = D_MODEL // _LANES


def _emb_body(x_hbm, table_hbm, out_hbm, idx_v, rows_v, *sems):
    gsems = list(sems[:_NBUF])
    ssems = list(sems[_NBUF:])
    sid = lax.axis_index("s")
    wid = sid * _NC + lax.axis_index("c")
    base = wid * _BPW

    pltpu.sync_copy(x_hbm.at[pl.ds(base, _BPW)], idx_v)

    def start_gather(c, b):
        return pltpu.async_copy(
            table_hbm.at[idx_v.at[pl.ds(c * _C, _C)]], rows_v.at[b], gsems[b])

    def start_scatter(c, b):
        return pltpu.async_copy(
            rows_v.at[b], out_hbm.at[pl.ds(base + c * _C, _C)], ssems[b])

    def scale_chunk(b):
        @plsc.parallel_loop(0, _C, 1, unroll=1)
        def _scale_rows(r):
            for j in range(_SLICES_PER_ROW):
                sl = pl.ds(j * _LANES, _LANES)
                rows_v[b, r, sl] = rows_v[b, r, sl] * SCALE

    gathers = [None] * _NCHUNK
    scatters = [None] * _NCHUNK
    for c in range(_NBUF - 1):
        gathers[c] = start_gather(c, c % _NBUF)

    for c in range(_NCHUNK):
        b = c % _NBUF
        gathers[c].wait()
        scale_chunk(b)
        scatters[c] = start_scatter(c, b)
        nc = c + _NBUF - 1
        if nc < _NCHUNK:
            if nc - _NBUF >= 0:
                scatters[nc - _NBUF].wait()
            gathers[nc] = start_gather(nc, nc % _NBUF)

    for c in range(_NCHUNK - _NBUF, _NCHUNK):
        scatters[c].wait()


@jax.jit
def kernel(x, table):
    xf = x.reshape(-1).astype(jnp.int32)

    mesh = plsc.VectorSubcoreMesh(core_axis_name="c", subcore_axis_name="s")
    run = functools.partial(
        pl.kernel,
        mesh=mesh,
        out_type=jax.ShapeDtypeStruct((_B, D_MODEL), jnp.float32),
        scratch_types=[
            pltpu.VMEM((_BPW,), jnp.int32),
            pltpu.VMEM((_NBUF, _C, D_MODEL), jnp.float32),
        ] + [pltpu.SemaphoreType.DMA] * (2 * _NBUF),
    )(_emb_body)
    out = run(xf, table)
    return out.reshape(x.shape + (D_MODEL,))


# speedup vs baseline: 1.0588x; 1.0016x over previous
import functools

import jax
import jax.numpy as jnp
from jax import lax
from jax.experimental import pallas as pl
from jax.experimental.pallas import tpu as pltpu
from jax.experimental.pallas import tpu_sc as plsc

D_MODEL = 1024
SCALE = 32.0

_NC = 2
_NS = 16
_NW = _NC * _NS

_ROWS = 4
_COLS = 4096
_B = _ROWS * _COLS
_BPW = _B // _NW
_C = 32
_NCHUNK = _BPW // _C
_NBUF = 3
_LANES = 16
_SLICES_PER_ROW = D_MODEL // _LANES


def _emb_body(x_hbm, table_hbm, out_hbm, idx_v, rows_v, *sems):
    gsems = list(sems[:_NBUF])
    ssems = list(sems[_NBUF:])
    wid = lax.axis_index("s") * _NC + lax.axis_index("c")
    xrow = wid // (_COLS // _BPW)
    xcol = (wid % (_COLS // _BPW)) * _BPW

    pltpu.sync_copy(x_hbm.at[xrow, pl.ds(xcol, _BPW)], idx_v)

    def start_gather(c, b):
        return pltpu.async_copy(
            table_hbm.at[idx_v.at[pl.ds(c * _C, _C)]], rows_v.at[b], gsems[b])

    def start_scatter(c, b):
        return pltpu.async_copy(
            rows_v.at[b], out_hbm.at[xrow, pl.ds(xcol + c * _C, _C)], ssems[b])

    def scale_chunk(b):
        @plsc.parallel_loop(0, _C, 1, unroll=1)
        def _scale_rows(r):
            for j in range(_SLICES_PER_ROW):
                sl = pl.ds(j * _LANES, _LANES)
                rows_v[b, r, sl] = rows_v[b, r, sl] * SCALE

    gathers = [None] * _NCHUNK
    scatters = [None] * _NCHUNK
    for c in range(_NBUF - 1):
        gathers[c] = start_gather(c, c % _NBUF)

    for c in range(_NCHUNK):
        b = c % _NBUF
        gathers[c].wait()
        scale_chunk(b)
        scatters[c] = start_scatter(c, b)
        nc = c + _NBUF - 1
        if nc < _NCHUNK:
            if nc - _NBUF >= 0:
                scatters[nc - _NBUF].wait()
            gathers[nc] = start_gather(nc, nc % _NBUF)

    for c in range(_NCHUNK - _NBUF, _NCHUNK):
        scatters[c].wait()


@jax.jit
def kernel(x, table):
    mesh = plsc.VectorSubcoreMesh(core_axis_name="c", subcore_axis_name="s")
    run = functools.partial(
        pl.kernel,
        mesh=mesh,
        out_type=jax.ShapeDtypeStruct((_ROWS, _COLS, D_MODEL), jnp.float32),
        scratch_types=[
            pltpu.VMEM((_BPW,), jnp.int32),
            pltpu.VMEM((_NBUF, _C, D_MODEL), jnp.float32),
        ] + [pltpu.SemaphoreType.DMA] * (2 * _NBUF),
    )(_emb_body)
    return run(x, table)


# speedup vs baseline: 1.2160x; 1.1485x over previous
import functools

import jax
import jax.numpy as jnp
from jax import lax
from jax.experimental import pallas as pl
from jax.experimental.pallas import tpu as pltpu
from jax.experimental.pallas import tpu_sc as plsc

D_MODEL = 1024
SCALE = 32.0

_NC = 2
_NS = 16
_NW = _NC * _NS

_ROWS = 4
_COLS = 4096
_B = _ROWS * _COLS
_BPW = _B // _NW
_C = 16
_NCHUNK = _BPW // _C
_NBUF = 6
_LANES = 16
_SLICES_PER_ROW = D_MODEL // _LANES


def _emb_body(x_hbm, table_hbm, out_hbm, idx_v, rows_v, gsem, ssem):
    wid = lax.axis_index("s") * _NC + lax.axis_index("c")
    xrow = wid // (_COLS // _BPW)
    xcol = (wid % (_COLS // _BPW)) * _BPW

    pltpu.sync_copy(x_hbm.at[xrow, pl.ds(xcol, _BPW)], idx_v)

    def start_gather(c, b):
        pltpu.async_copy(
            table_hbm.at[idx_v.at[pl.ds(c * _C, _C)]], rows_v.at[b],
            gsem.at[b])

    def wait_gather(b):
        pltpu.make_async_copy(
            table_hbm.at[idx_v.at[pl.ds(0, _C)]], rows_v.at[b],
            gsem.at[b]).wait()

    def start_scatter(c, b):
        pltpu.async_copy(
            rows_v.at[b], out_hbm.at[xrow, pl.ds(xcol + c * _C, _C)],
            ssem.at[b])

    def wait_scatter(b):
        pltpu.make_async_copy(
            rows_v.at[b], out_hbm.at[0, pl.ds(0, _C)], ssem.at[b]).wait()

    def scale_chunk(b):
        @plsc.parallel_loop(0, _C, 1, unroll=1)
        def _scale_rows(r):
            for j in range(_SLICES_PER_ROW):
                sl = pl.ds(j * _LANES, _LANES)
                rows_v[b, r, sl] = rows_v[b, r, sl] * SCALE

    for c in range(_NBUF - 1):
        start_gather(c, c)

    def chunk_body(c, carry):
        b = lax.rem(c, _NBUF)
        wait_gather(b)
        scale_chunk(b)
        start_scatter(c, b)
        nc = c + _NBUF - 1
        @pl.when(nc < _NCHUNK)
        def _():
            nb = lax.rem(nc, _NBUF)
            @pl.when(nc >= _NBUF)
            def _():
                wait_scatter(nb)
            start_gather(nc, nb)
        return carry

    lax.fori_loop(0, _NCHUNK, chunk_body, 0)

    def drain_body(c, carry):
        wait_scatter(lax.rem(c, _NBUF))
        return carry

    lax.fori_loop(_NCHUNK - _NBUF, _NCHUNK, drain_body, 0)


@jax.jit
def kernel(x, table):
    mesh = plsc.VectorSubcoreMesh(core_axis_name="c", subcore_axis_name="s")
    run = functools.partial(
        pl.kernel,
        mesh=mesh,
        out_type=jax.ShapeDtypeStruct((_ROWS, _COLS, D_MODEL), jnp.float32),
        scratch_types=[
            pltpu.VMEM((_BPW,), jnp.int32),
            pltpu.VMEM((_NBUF, _C, D_MODEL), jnp.float32),
            pltpu.SemaphoreType.DMA((_NBUF,)),
            pltpu.SemaphoreType.DMA((_NBUF,)),
        ],
        compiler_params=pltpu.CompilerParams(
            disable_bounds_checks=True,
            disable_semaphore_checks=True,
            skip_device_barrier=True,
        ),
    )(_emb_body)
    return run(x, table)


# speedup vs baseline: 1.2322x; 1.0133x over previous
import functools

import jax
import jax.numpy as jnp
from jax import lax
from jax.experimental import pallas as pl
from jax.experimental.pallas import tpu as pltpu
from jax.experimental.pallas import tpu_sc as plsc

D_MODEL = 1024
SCALE = 32.0

_NC = 2
_NS = 16
_NW = _NC * _NS

_ROWS = 4
_COLS = 4096
_B = _ROWS * _COLS
_BPW = _B // _NW
_C = 16
_NCHUNK = _BPW // _C
_NBUF = 7
_LANES = 16
_SLICES_PER_ROW = D_MODEL // _LANES


def _emb_body(x_hbm, table_hbm, out_hbm, idx_v, rows_v, gsem, ssem):
    wid = lax.axis_index("s") * _NC + lax.axis_index("c")
    xrow = wid // (_COLS // _BPW)
    xcol = (wid % (_COLS // _BPW)) * _BPW

    pltpu.sync_copy(x_hbm.at[xrow, pl.ds(xcol, _BPW)], idx_v)

    def start_gather(c, b):
        pltpu.async_copy(
            table_hbm.at[idx_v.at[pl.ds(c * _C, _C)]], rows_v.at[b],
            gsem.at[b])

    def wait_gather(b):
        pltpu.make_async_copy(
            table_hbm.at[idx_v.at[pl.ds(0, _C)]], rows_v.at[b],
            gsem.at[b]).wait()

    def start_scatter(c, b):
        pltpu.async_copy(
            rows_v.at[b], out_hbm.at[xrow, pl.ds(xcol + c * _C, _C)],
            ssem.at[b])

    def wait_scatter(b):
        pltpu.make_async_copy(
            rows_v.at[b], out_hbm.at[0, pl.ds(0, _C)], ssem.at[b]).wait()

    def scale_chunk(b):
        @plsc.parallel_loop(0, _C, 1, unroll=1)
        def _scale_rows(r):
            for j in range(_SLICES_PER_ROW):
                sl = pl.ds(j * _LANES, _LANES)
                rows_v[b, r, sl] = rows_v[b, r, sl] * SCALE

    for c in range(_NBUF - 1):
        start_gather(c, c)

    def chunk_body(c, carry):
        b = lax.rem(c, _NBUF)
        wait_gather(b)
        scale_chunk(b)
        start_scatter(c, b)
        nc = c + _NBUF - 1
        @pl.when(nc < _NCHUNK)
        def _():
            nb = lax.rem(nc, _NBUF)
            @pl.when(nc >= _NBUF)
            def _():
                wait_scatter(nb)
            start_gather(nc, nb)
        return carry

    lax.fori_loop(0, _NCHUNK, chunk_body, 0)

    def drain_body(c, carry):
        wait_scatter(lax.rem(c, _NBUF))
        return carry

    lax.fori_loop(_NCHUNK - _NBUF, _NCHUNK, drain_body, 0)


@jax.jit
def kernel(x, table):
    mesh = plsc.VectorSubcoreMesh(core_axis_name="c", subcore_axis_name="s")
    run = functools.partial(
        pl.kernel,
        mesh=mesh,
        out_type=jax.ShapeDtypeStruct((_ROWS, _COLS, D_MODEL), jnp.float32),
        scratch_types=[
            pltpu.VMEM((_BPW,), jnp.int32),
            pltpu.VMEM((_NBUF, _C, D_MODEL), jnp.float32),
            pltpu.SemaphoreType.DMA((_NBUF,)),
            pltpu.SemaphoreType.DMA((_NBUF,)),
        ],
        compiler_params=pltpu.CompilerParams(
            disable_bounds_checks=True,
            disable_semaphore_checks=True,
            skip_device_barrier=True,
        ),
    )(_emb_body)
    return run(x, table)
